# baseline (device time: 257031 ns/iter reference)
import jax
import jax.numpy as jnp
from jax import lax
from jax.experimental import pallas as pl
from jax.experimental.pallas import tpu as pltpu

M = 2048
D = 2048
M_HALF = M // 2


def _allreduce_body(partial_ref, out_ref, recv_y_ref, sems):
    my_x = lax.axis_index("x")
    my_y = lax.axis_index("y")
    y_peer = (my_x, 1 - my_y)
    x_peer = (1 - my_x, my_y)

    barrier = pltpu.get_barrier_semaphore()
    for peer in (y_peer, x_peer):
        pl.semaphore_signal(
            barrier, inc=1, device_id=peer,
            device_id_type=pl.DeviceIdType.MESH,
        )
    pl.semaphore_wait(barrier, 2)

    rdma_y = pltpu.make_async_remote_copy(
        src_ref=partial_ref,
        dst_ref=recv_y_ref,
        send_sem=sems.at[0],
        recv_sem=sems.at[1],
        device_id=y_peer,
        device_id_type=pl.DeviceIdType.MESH,
    )
    rdma_y.start()
    rdma_y.wait()

    row0 = my_x * M_HALF
    out_ref[pl.ds(row0, M_HALF), :] = partial_ref[...] + recv_y_ref[...]

    rdma_x = pltpu.make_async_remote_copy(
        src_ref=out_ref.at[pl.ds(row0, M_HALF), :],
        dst_ref=out_ref.at[pl.ds(row0, M_HALF), :],
        send_sem=sems.at[2],
        recv_sem=sems.at[3],
        device_id=x_peer,
        device_id_type=pl.DeviceIdType.MESH,
    )
    rdma_x.start()
    rdma_x.wait()


def kernel(dy, W):
    my_x = lax.axis_index("x")
    dy_half = lax.dynamic_slice_in_dim(dy, my_x * M_HALF, M_HALF, axis=0)
    partial = lax.dot_general(
        dy_half, W,
        dimension_numbers=(((1,), (1,)), ((), ())),
        preferred_element_type=jnp.float32,
    )
    return pl.pallas_call(
        _allreduce_body,
        out_shape=jax.ShapeDtypeStruct((M, D), jnp.float32),
        in_specs=[pl.BlockSpec(memory_space=pltpu.VMEM)],
        out_specs=pl.BlockSpec(memory_space=pltpu.VMEM),
        scratch_shapes=[
            pltpu.VMEM((M_HALF, D), jnp.float32),
            pltpu.SemaphoreType.DMA((4,)),
        ],
        compiler_params=pltpu.CompilerParams(collective_id=0),
    )(partial)


# device time: 178154 ns/iter; 1.4427x vs baseline; 1.4427x over previous
import jax
import jax.numpy as jnp
from jax import lax
from jax.experimental import pallas as pl
from jax.experimental.pallas import tpu as pltpu

M = 2048
D = 2048
M_HALF = M // 2


C = 8
R = M_HALF // C


def _allreduce_body(partial_ref, out_ref, recv_y_ref,
                    y_send, y_recv, x_send, x_recv):
    my_x = lax.axis_index("x")
    my_y = lax.axis_index("y")
    y_peer = (my_x, 1 - my_y)
    x_peer = (1 - my_x, my_y)

    barrier = pltpu.get_barrier_semaphore()
    for peer in (y_peer, x_peer):
        pl.semaphore_signal(
            barrier, inc=1, device_id=peer,
            device_id_type=pl.DeviceIdType.MESH,
        )
    pl.semaphore_wait(barrier, 2)

    row0 = my_x * M_HALF

    rdmas_y = []
    for j in range(C):
        r = pltpu.make_async_remote_copy(
            src_ref=partial_ref.at[pl.ds(j * R, R)],
            dst_ref=recv_y_ref.at[pl.ds(j * R, R)],
            send_sem=y_send.at[j],
            recv_sem=y_recv.at[j],
            device_id=y_peer,
            device_id_type=pl.DeviceIdType.MESH,
        )
        r.start()
        rdmas_y.append(r)

    rdmas_x = []
    for j in range(C):
        rdmas_y[j].wait_recv()
        out_ref[pl.ds(row0 + j * R, R), :] = (
            partial_ref[pl.ds(j * R, R), :] + recv_y_ref[pl.ds(j * R, R), :]
        )
        rx = pltpu.make_async_remote_copy(
            src_ref=out_ref.at[pl.ds(row0 + j * R, R)],
            dst_ref=out_ref.at[pl.ds(row0 + j * R, R)],
            send_sem=x_send.at[j],
            recv_sem=x_recv.at[j],
            device_id=x_peer,
            device_id_type=pl.DeviceIdType.MESH,
        )
        rx.start()
        rdmas_x.append(rx)

    for j in range(C):
        rdmas_x[j].wait_recv()
    for j in range(C):
        rdmas_y[j].wait_send()
        rdmas_x[j].wait_send()


def kernel(dy, W):
    my_x = lax.axis_index("x")
    dy_half = lax.dynamic_slice_in_dim(dy, my_x * M_HALF, M_HALF, axis=0)
    partial = lax.dot_general(
        dy_half, W,
        dimension_numbers=(((1,), (1,)), ((), ())),
        preferred_element_type=jnp.float32,
    )
    return pl.pallas_call(
        _allreduce_body,
        out_shape=jax.ShapeDtypeStruct((M, D), jnp.float32),
        in_specs=[pl.BlockSpec(memory_space=pltpu.VMEM)],
        out_specs=pl.BlockSpec(memory_space=pltpu.VMEM),
        scratch_shapes=[
            pltpu.VMEM((M_HALF, D), jnp.float32),
            pltpu.SemaphoreType.DMA((C,)),
            pltpu.SemaphoreType.DMA((C,)),
            pltpu.SemaphoreType.DMA((C,)),
            pltpu.SemaphoreType.DMA((C,)),
        ],
        compiler_params=pltpu.CompilerParams(collective_id=0),
    )(partial)
